# baseline (speedup 1.0000x reference)
import jax
import jax.numpy as jnp
from jax import lax
from jax.experimental import pallas as pl
from jax.experimental.pallas import tpu as pltpu
from jax.experimental.pallas import tpu_sc as plsc

B = 16384
D = 64
NC = 2
NS = 16
NW = NC * NS
BPW = B // NW
CHUNK = 128
NCHUNK = BPW // CHUNK


def _body(user_hbm, item_p_hbm, item_n_hbm, uf_hbm, if_hbm, out_hbm,
          idx_u, idx_p, idx_n, u_rows, p_rows, n_rows, out_v,
          sem_u, sem_p, sem_n):
  wid = lax.axis_index("s") * NC + lax.axis_index("c")
  base = wid * BPW

  for j in range(NCHUNK):
    pltpu.sync_copy(user_hbm.at[pl.ds(base + j * CHUNK, CHUNK)], idx_u.at[j])
    pltpu.sync_copy(item_p_hbm.at[pl.ds(base + j * CHUNK, CHUNK)], idx_p.at[j])
    pltpu.sync_copy(item_n_hbm.at[pl.ds(base + j * CHUNK, CHUNK)], idx_n.at[j])

  copies = []
  for j in range(NCHUNK):
    sl = pl.ds(j * CHUNK, CHUNK)
    copies.append(pltpu.async_copy(uf_hbm.at[idx_u.at[j]], u_rows.at[sl], sem_u))
    copies.append(pltpu.async_copy(if_hbm.at[idx_p.at[j]], p_rows.at[sl], sem_p))
    copies.append(pltpu.async_copy(if_hbm.at[idx_n.at[j]], n_rows.at[sl], sem_n))
  for c in copies:
    c.wait()

  lane = lax.iota(jnp.int32, 16)

  def blk(b, carry):
    r0 = b * 16
    row_idx = r0 + lane
    acc = jnp.zeros((16,), jnp.float32)
    for d in range(D):
      col = jnp.full((16,), d, jnp.int32)
      un = plsc.load_gather(u_rows, [row_idx, col])
      pn = plsc.load_gather(p_rows, [row_idx, col])
      nn = plsc.load_gather(n_rows, [row_idx, col])
      acc = acc + un * (pn - nn)
    out_v[pl.ds(r0, 16)] = acc
    return carry

  lax.fori_loop(0, BPW // 16, blk, 0)
  pltpu.sync_copy(out_v, out_hbm.at[pl.ds(base, BPW)])


@jax.jit
def kernel(user, item_p, item_n, user_factors, item_factors):
  mesh = plsc.VectorSubcoreMesh(
      core_axis_name="c", subcore_axis_name="s",
      num_cores=NC, num_subcores=NS)
  k = pl.kernel(
      _body,
      out_type=jax.ShapeDtypeStruct((B,), jnp.float32),
      mesh=mesh,
      compiler_params=pltpu.CompilerParams(
          needs_layout_passes=False, use_tc_tiling_on_sc=False),
      scratch_types=[
          pltpu.VMEM((NCHUNK, CHUNK), jnp.int32),
          pltpu.VMEM((NCHUNK, CHUNK), jnp.int32),
          pltpu.VMEM((NCHUNK, CHUNK), jnp.int32),
          pltpu.VMEM((BPW, D), jnp.float32),
          pltpu.VMEM((BPW, D), jnp.float32),
          pltpu.VMEM((BPW, D), jnp.float32),
          pltpu.VMEM((BPW,), jnp.float32),
          pltpu.SemaphoreType.DMA,
          pltpu.SemaphoreType.DMA,
          pltpu.SemaphoreType.DMA,
      ],
  )
  return k(user, item_p, item_n, user_factors, item_factors)


# speedup vs baseline: 1.6440x; 1.6440x over previous
import jax
import jax.numpy as jnp
from jax import lax
from jax.experimental import pallas as pl
from jax.experimental.pallas import tpu as pltpu
from jax.experimental.pallas import tpu_sc as plsc

B = 16384
D = 64
NC = 2
NS = 16
NW = NC * NS
BPW = B // NW
CHUNK = 128
NCHUNK = BPW // CHUNK

_SCR = 17 * 16


def _body(user_hbm, item_p_hbm, item_n_hbm, uf_hbm, if_hbm, out_hbm,
          idx_u, idx_p, idx_n, u_rows, p_rows, n_rows, out_v, scr,
          sem_u, sem_p, sem_n):
  wid = lax.axis_index("s") * NC + lax.axis_index("c")
  base = wid * BPW

  pltpu.sync_copy(user_hbm.at[pl.ds(base, BPW)], idx_u)
  pltpu.sync_copy(item_p_hbm.at[pl.ds(base, BPW)], idx_p)
  pltpu.sync_copy(item_n_hbm.at[pl.ds(base, BPW)], idx_n)

  lane = lax.iota(jnp.int32, 16)
  lane17 = lane * 17

  def chunk_step(c, carry):
    c0 = c * CHUNK

    def fetch(g, carry2):
      g0 = g * 16
      iu = idx_u[pl.ds(c0 + g0, 16)]
      ip = idx_p[pl.ds(c0 + g0, 16)]
      inn = idx_n[pl.ds(c0 + g0, 16)]
      for j in range(16):
        pltpu.async_copy(uf_hbm.at[iu[j]], u_rows.at[g0 + j], sem_u)
        pltpu.async_copy(if_hbm.at[ip[j]], p_rows.at[g0 + j], sem_p)
        pltpu.async_copy(if_hbm.at[inn[j]], n_rows.at[g0 + j], sem_n)
      return carry2

    lax.fori_loop(0, CHUNK // 16, fetch, 0)

    def drain(i, carry2):
      pltpu.make_async_copy(uf_hbm.at[0], u_rows.at[i], sem_u).wait()
      pltpu.make_async_copy(if_hbm.at[0], p_rows.at[i], sem_p).wait()
      pltpu.make_async_copy(if_hbm.at[0], n_rows.at[i], sem_n).wait()
      return carry2

    lax.fori_loop(0, CHUNK, drain, 0)

    def blk(bi, carry2):
      r0 = bi * 16
      for j in range(16):
        r = r0 + j
        acc = None
        for q in range(D // 16):
          sl = pl.ds(q * 16, 16)
          t = u_rows[r, sl] * (p_rows[r, sl] - n_rows[r, sl])
          acc = t if acc is None else acc + t
        plsc.store_scatter(scr, [lane17 + j], acc)
      tot = None
      for d in range(16):
        v = scr[pl.ds(d * 17, 16)]
        tot = v if tot is None else tot + v
      out_v[pl.ds(c0 + r0, 16)] = tot
      return carry2

    lax.fori_loop(0, CHUNK // 16, blk, 0)
    return carry

  lax.fori_loop(0, NCHUNK, chunk_step, 0)
  pltpu.sync_copy(out_v, out_hbm.at[pl.ds(base, BPW)])


@jax.jit
def kernel(user, item_p, item_n, user_factors, item_factors):
  mesh = plsc.VectorSubcoreMesh(
      core_axis_name="c", subcore_axis_name="s",
      num_cores=NC, num_subcores=NS)
  k = pl.kernel(
      _body,
      out_type=jax.ShapeDtypeStruct((B,), jnp.float32),
      mesh=mesh,
      compiler_params=pltpu.CompilerParams(needs_layout_passes=False),
      scratch_types=[
          pltpu.VMEM((BPW,), jnp.int32),
          pltpu.VMEM((BPW,), jnp.int32),
          pltpu.VMEM((BPW,), jnp.int32),
          pltpu.VMEM((CHUNK, D), jnp.float32),
          pltpu.VMEM((CHUNK, D), jnp.float32),
          pltpu.VMEM((CHUNK, D), jnp.float32),
          pltpu.VMEM((BPW,), jnp.float32),
          pltpu.VMEM((_SCR,), jnp.float32),
          pltpu.SemaphoreType.DMA,
          pltpu.SemaphoreType.DMA,
          pltpu.SemaphoreType.DMA,
      ],
  )
  return k(user, item_p, item_n, user_factors, item_factors)


# speedup vs baseline: 2.2254x; 1.3536x over previous
import jax
import jax.numpy as jnp
from jax import lax
from jax.experimental import pallas as pl
from jax.experimental.pallas import tpu as pltpu
from jax.experimental.pallas import tpu_sc as plsc

B = 16384
D = 64
NC = 2
NS = 16
NW = NC * NS
BPW = B // NW
CH = 16
NCH = BPW // CH
TPR = 8

_SCR = 17 * 16


def _body(user_hbm, item_p_hbm, item_n_hbm, uf3, if3, out_hbm,
          idx_u, idx_p, idx_n,
          u_t0, p_t0, n_t0, u_t1, p_t1, n_t1,
          out_v, scr,
          sem_u0, sem_p0, sem_n0, sem_u1, sem_p1, sem_n1):
  wid = lax.axis_index("s") * NC + lax.axis_index("c")
  base = wid * BPW

  pltpu.sync_copy(user_hbm.at[pl.ds(base, BPW)], idx_u)
  pltpu.sync_copy(item_p_hbm.at[pl.ds(base, BPW)], idx_p)
  pltpu.sync_copy(item_n_hbm.at[pl.ds(base, BPW)], idx_n)

  lane = lax.iota(jnp.int32, 16)
  lane17 = lane * 17

  bufs = ((u_t0, p_t0, n_t0), (u_t1, p_t1, n_t1))
  sems = ((sem_u0, sem_p0, sem_n0), (sem_u1, sem_p1, sem_n1))

  def load_idx(c):
    c0 = c * CH
    return (idx_u[pl.ds(c0, CH)], idx_p[pl.ds(c0, CH)], idx_n[pl.ds(c0, CH)])

  def fire(idxs, slot):
    for (src, buf, sem), iv in zip(
        ((uf3, bufs[slot][0], sems[slot][0]),
         (if3, bufs[slot][1], sems[slot][1]),
         (if3, bufs[slot][2], sems[slot][2])), idxs):
      tv = iv >> 3
      for j in range(CH):
        pltpu.async_copy(src.at[tv[j]], buf.at[j], sem)

  def wait(slot):
    for src, buf, sem in (
        (uf3, bufs[slot][0], sems[slot][0]),
        (if3, bufs[slot][1], sems[slot][1]),
        (if3, bufs[slot][2], sems[slot][2])):
      for j in range(CH):
        pltpu.make_async_copy(src.at[0], buf.at[j], sem).wait()

  def compute(c, idxs, slot):
    ub, pb, nb = bufs[slot]
    iu, ip, inn = idxs
    for j in range(CH):
      qu = iu[j] & (TPR - 1)
      qp = ip[j] & (TPR - 1)
      qn = inn[j] & (TPR - 1)
      acc = None
      for q in range(D // 16):
        sl = pl.ds(q * 16, 16)
        t = ub[j, qu, sl] * (pb[j, qp, sl] - nb[j, qn, sl])
        acc = t if acc is None else acc + t
      plsc.store_scatter(scr, [lane17 + j], acc)
    tot = None
    for d in range(16):
      v = scr[pl.ds(d * 17, 16)]
      tot = v if tot is None else tot + v
    out_v[pl.ds(c * CH, 16)] = tot

  fire(load_idx(0), 0)

  def pair(pi, carry):
    c0 = 2 * pi
    cur0 = carry
    cur1 = load_idx(c0 + 1)
    fire(cur1, 1)
    wait(0)
    compute(c0, cur0, 0)
    nxt0 = load_idx(jnp.minimum(c0 + 2, NCH - 2))

    @pl.when(pi < NCH // 2 - 1)
    def _():
      fire(nxt0, 0)

    wait(1)
    compute(c0 + 1, cur1, 1)
    return nxt0

  lax.fori_loop(0, NCH // 2, pair, load_idx(0))
  pltpu.sync_copy(out_v, out_hbm.at[pl.ds(base, BPW)])


@jax.jit
def kernel(user, item_p, item_n, user_factors, item_factors):
  uf3 = user_factors.reshape(user_factors.shape[0] // TPR, TPR, D)
  if3 = item_factors.reshape(item_factors.shape[0] // TPR, TPR, D)
  mesh = plsc.VectorSubcoreMesh(
      core_axis_name="c", subcore_axis_name="s",
      num_cores=NC, num_subcores=NS)
  tile = pltpu.VMEM((CH, TPR, D), jnp.float32)
  k = pl.kernel(
      _body,
      out_type=jax.ShapeDtypeStruct((B,), jnp.float32),
      mesh=mesh,
      compiler_params=pltpu.CompilerParams(needs_layout_passes=False),
      scratch_types=[
          pltpu.VMEM((BPW,), jnp.int32),
          pltpu.VMEM((BPW,), jnp.int32),
          pltpu.VMEM((BPW,), jnp.int32),
          tile, tile, tile, tile, tile, tile,
          pltpu.VMEM((BPW,), jnp.float32),
          pltpu.VMEM((_SCR,), jnp.float32),
          pltpu.SemaphoreType.DMA,
          pltpu.SemaphoreType.DMA,
          pltpu.SemaphoreType.DMA,
          pltpu.SemaphoreType.DMA,
          pltpu.SemaphoreType.DMA,
          pltpu.SemaphoreType.DMA,
      ],
  )
  return k(user, item_p, item_n, uf3, if3)
